# baseline (speedup 1.0000x reference)
import functools

import jax
import jax.numpy as jnp
from jax import lax
from jax.experimental import pallas as pl
from jax.experimental.pallas import tpu as pltpu
from jax.experimental.pallas import tpu_sc as plsc

_NC = 2
_NS = 16
_NW = _NC * _NS
_GB = 128
_NBUF = 4


def _sc_gather(n_chunks, n_rows, d, dp):
    n_outer = n_chunks // _NBUF
    mesh = plsc.VectorSubcoreMesh(core_axis_name="c", subcore_axis_name="s")

    @functools.partial(
        pl.kernel,
        out_type=jax.ShapeDtypeStruct((n_rows, dp), jnp.float32),
        mesh=mesh,
        scratch_types=[
            pltpu.VMEM((n_chunks, _GB), jnp.int32),
            pltpu.VMEM((_NBUF, _GB, dp), jnp.float32),
            pltpu.SemaphoreType.DMA((_NBUF,)),
            pltpu.SemaphoreType.DMA,
        ],
        compiler_params=pltpu.CompilerParams(use_tc_tiling_on_sc=True),
    )
    def emb(idx_hbm, table_hbm, out_hbm, idx_v, rows_v, gsem, osem):
        wid = lax.axis_index("s") * _NC + lax.axis_index("c")
        pltpu.sync_copy(idx_hbm.at[wid], idx_v)
        base = wid * n_chunks

        def fire(chunk, slot):
            pltpu.async_copy(
                table_hbm.at[idx_v.at[chunk]], rows_v.at[slot], gsem.at[slot]
            )

        def drain(chunk, slot):
            pltpu.make_async_copy(
                table_hbm.at[idx_v.at[chunk]], rows_v.at[slot], gsem.at[slot]
            ).wait()
            src = rows_v.at[slot]
            dst = out_hbm.at[pl.ds((base + chunk) * _GB, _GB)]
            pltpu.async_copy(src, dst, osem)
            pltpu.make_async_copy(src, dst, osem).wait()

        for b in range(_NBUF):
            fire(b, b)

        @pl.loop(0, n_outer - 1)
        def _(i):
            for b in range(_NBUF):
                g = i * _NBUF + b
                drain(g, b)
                fire(g + _NBUF, b)

        for b in range(_NBUF):
            drain((n_outer - 1) * _NBUF + b, b)

    return emb


def kernel(input, table):
    bsz, nf = input.shape
    v, d = table.shape
    tot = bsz * nf
    group = _NW * _GB * _NBUF
    tot_p = ((tot + group - 1) // group) * group
    flat = input.reshape(-1).astype(jnp.int32)
    if tot_p != tot:
        flat = jnp.concatenate([flat, jnp.zeros((tot_p - tot,), jnp.int32)])
    n_chunks = tot_p // (_NW * _GB)
    idx = flat.reshape(_NW, n_chunks, _GB)
    dp = 128
    tbl = jnp.pad(table, ((0, 0), (0, dp - d))) if d != dp else table
    out = _sc_gather(n_chunks, tot_p, d, dp)(idx, tbl)
    return out[:tot, :d].reshape(bsz, nf, d)


# speedup vs baseline: 1.3030x; 1.3030x over previous
import functools

import jax
import jax.numpy as jnp
from jax import lax
from jax.experimental import pallas as pl
from jax.experimental.pallas import tpu as pltpu
from jax.experimental.pallas import tpu_sc as plsc

_NC = 2
_NS = 16
_NW = _NC * _NS
_NBUF = 4


def _sc_gather(n_chunks, gb, nf, nfp, d, dp, n_rows):
    bpc = gb // nf
    n_outer = n_chunks // _NBUF
    mesh = plsc.VectorSubcoreMesh(core_axis_name="c", subcore_axis_name="s")

    @functools.partial(
        pl.kernel,
        out_type=jax.ShapeDtypeStruct((n_rows, dp), jnp.float32),
        mesh=mesh,
        scratch_types=[
            pltpu.VMEM((n_chunks, gb), jnp.int32),
            pltpu.VMEM((_NBUF, gb, d), jnp.float32),
            pltpu.SemaphoreType.DMA((_NBUF,)),
            pltpu.SemaphoreType.DMA((_NBUF,)),
        ],
        compiler_params=pltpu.CompilerParams(use_tc_tiling_on_sc=False),
    )
    def emb(idx_hbm, table_hbm, out_hbm, idx_v, rows_v, gsem, wsem):
        wid = lax.axis_index("s") * _NC + lax.axis_index("c")
        pltpu.sync_copy(idx_hbm.at[wid], idx_v)
        base = wid * n_chunks

        def wb_pair(chunk, slot, k):
            src = rows_v.at[slot, pl.ds(k * nf, nf)]
            b = (base + chunk) * bpc + k
            dst = out_hbm.at[pl.ds(b * nfp, nf), pl.ds(0, d)]
            return src, dst

        def fire(chunk, slot):
            pltpu.async_copy(
                table_hbm.at[idx_v.at[chunk]], rows_v.at[slot], gsem.at[slot]
            )

        def drain(chunk, slot):
            pltpu.make_async_copy(
                table_hbm.at[idx_v.at[chunk]], rows_v.at[slot], gsem.at[slot]
            ).wait()
            for k in range(bpc):
                src, dst = wb_pair(chunk, slot, k)
                pltpu.async_copy(src, dst, wsem.at[slot])

        def wait_wb(chunk, slot):
            for k in range(bpc):
                src, dst = wb_pair(chunk, slot, k)
                pltpu.make_async_copy(src, dst, wsem.at[slot]).wait()

        for b in range(_NBUF):
            fire(b, b)

        @pl.loop(0, n_outer - 1)
        def _(i):
            for b in range(_NBUF):
                g = i * _NBUF + b
                drain(g, b)
                wait_wb(g, b)
                fire(g + _NBUF, b)

        for b in range(_NBUF):
            g = (n_outer - 1) * _NBUF + b
            drain(g, b)
            wait_wb(g, b)

    return emb


def kernel(input, table):
    bsz, nf = input.shape
    v, d = table.shape
    nfp = ((nf + 7) // 8) * 8
    dp = 128
    bpc = 4
    gb = bpc * nf
    assert (bsz * nf) % (_NW * gb) == 0 and bsz % bpc == 0
    n_chunks = (bsz * nf) // (_NW * gb)
    assert n_chunks % _NBUF == 0, (n_chunks, _NBUF)
    flat = input.reshape(-1).astype(jnp.int32)
    idx = flat.reshape(_NW, n_chunks, gb)
    t2 = lax.optimization_barrier(table.reshape(v // 2, 2 * d))
    tlin = t2.reshape(v, d)
    out = _sc_gather(n_chunks, gb, nf, nfp, d, dp, bsz * nfp)(idx, tlin)
    return out.reshape(bsz, nfp, dp)[:, :nf, :d]
